# baseline (speedup 1.0000x reference)
import jax
import jax.numpy as jnp
from jax.experimental import pallas as pl
from jax.experimental.pallas import tpu as pltpu

D_MODEL = 768
NUM_HEADS = 12
U = 64
ROW_BLOCK = 512
KEY_CHUNK = 512


def _qkv_kernel(x_ref, wq_ref, wk_ref, wv_ref, bq_ref, bk_ref, bv_ref,
                q_ref, k_ref, v_ref):
    x = x_ref[...]
    q_ref[...] = jnp.dot(x, wq_ref[...], preferred_element_type=jnp.float32) + bq_ref[...]
    k_ref[...] = jnp.dot(x, wk_ref[...], preferred_element_type=jnp.float32) + bk_ref[...]
    v_ref[...] = jnp.dot(x, wv_ref[...], preferred_element_type=jnp.float32) + bv_ref[...]


def _attn_kernel(q_ref, k_ref, v_ref, o_ref, p_ref):
    Q = q_ref[0]
    K = k_ref[0]
    V = v_ref[0]
    T = Q.shape[0]

    nq = Q * jax.lax.rsqrt(jnp.maximum(jnp.sum(Q * Q, axis=-1, keepdims=True), 1e-12))
    nk = K * jax.lax.rsqrt(jnp.maximum(jnp.sum(K * K, axis=-1, keepdims=True), 1e-12))

    m = jnp.full((1, T), -jnp.inf, dtype=jnp.float32)
    for c in range(T // KEY_CHUNK):
        kc = nk[c * KEY_CHUNK:(c + 1) * KEY_CHUNK]
        st = jax.lax.dot_general(kc, nq, (((1,), (1,)), ((), ())),
                                 preferred_element_type=jnp.float32)
        m = jnp.maximum(m, jnp.max(st, axis=0, keepdims=True))

    iota = jax.lax.broadcasted_iota(jnp.int32, (1, T), 1)

    def body(r, v):
        mx = jnp.max(v)
        oh = v == mx
        idx = jnp.max(jnp.where(oh, iota, -1))
        sel = iota == idx
        p_ref[pl.ds(r, 1), :] = sel.astype(jnp.float32)
        return jnp.where(sel, -jnp.inf, v)

    jax.lax.fori_loop(0, U, body, m)

    P = p_ref[...]
    q_sel = jnp.dot(P, Q, preferred_element_type=jnp.float32)
    s = jax.lax.dot_general(q_sel, K, (((1,), (1,)), ((), ())),
                            preferred_element_type=jnp.float32) * 0.125
    s = s - jnp.max(s, axis=-1, keepdims=True)
    e = jnp.exp(s)
    a = e / jnp.sum(e, axis=-1, keepdims=True)
    out = jnp.dot(a, V, preferred_element_type=jnp.float32)
    o_ref[0] = jax.lax.dot_general(P, out, (((0,), (0,)), ((), ())),
                                   preferred_element_type=jnp.float32)


def _proj_kernel(o_ref, wo_ref, bo_ref, y_ref):
    y_ref[...] = jnp.dot(o_ref[...], wo_ref[...], preferred_element_type=jnp.float32) + bo_ref[...]


def kernel(x, Wq, bq, Wk, bk, Wv, bv, Wo, bo):
    B, T, d_model = x.shape
    H = NUM_HEADS
    d_k = d_model // H
    BT = B * T

    x2 = x.reshape(BT, d_model)
    bq2 = bq.reshape(1, d_model)
    bk2 = bk.reshape(1, d_model)
    bv2 = bv.reshape(1, d_model)
    bo2 = bo.reshape(1, d_model)

    n_row = BT // ROW_BLOCK
    row_spec = pl.BlockSpec((ROW_BLOCK, d_model), lambda i: (i, 0))
    w_spec = pl.BlockSpec((d_model, d_model), lambda i: (0, 0))
    b_spec = pl.BlockSpec((1, d_model), lambda i: (0, 0))

    q, k, v = pl.pallas_call(
        _qkv_kernel,
        grid=(n_row,),
        in_specs=[row_spec, w_spec, w_spec, w_spec, b_spec, b_spec, b_spec],
        out_specs=[row_spec, row_spec, row_spec],
        out_shape=[jax.ShapeDtypeStruct((BT, d_model), jnp.float32)] * 3,
    )(x2, Wq, Wk, Wv, bq2, bk2, bv2)

    BH = B * H
    q3 = q.reshape(BH, T, d_k)
    k3 = k.reshape(BH, T, d_k)
    v3 = v.reshape(BH, T, d_k)

    bh_spec = pl.BlockSpec((1, T, d_k), lambda j: (j, 0, 0))
    o = pl.pallas_call(
        _attn_kernel,
        grid=(BH,),
        in_specs=[bh_spec, bh_spec, bh_spec],
        out_specs=bh_spec,
        out_shape=jax.ShapeDtypeStruct((BH, T, d_k), jnp.float32),
        scratch_shapes=[pltpu.VMEM((U, T), jnp.float32)],
    )(q3, k3, v3)

    o2 = o.reshape(BT, d_model)
    y = pl.pallas_call(
        _proj_kernel,
        grid=(n_row,),
        in_specs=[row_spec, w_spec, b_spec],
        out_specs=row_spec,
        out_shape=jax.ShapeDtypeStruct((BT, d_model), jnp.float32),
    )(o2, Wo, bo2)

    return y.reshape(B, T, d_model)


# speedup vs baseline: 2.8037x; 2.8037x over previous
import jax
import jax.numpy as jnp
from jax.experimental import pallas as pl
from jax.experimental.pallas import tpu as pltpu

D_MODEL = 768
NUM_HEADS = 12
U = 64
ROW_BLOCK = 512
KEY_CHUNK = 512


def _qkv_kernel(x_ref, wq_ref, wk_ref, wv_ref, bq_ref, bk_ref, bv_ref,
                q_ref, k_ref, v_ref):
    x = x_ref[...]
    q_ref[...] = jnp.dot(x, wq_ref[...], preferred_element_type=jnp.float32) + bq_ref[...]
    k_ref[...] = jnp.dot(x, wk_ref[...], preferred_element_type=jnp.float32) + bk_ref[...]
    v_ref[...] = jnp.dot(x, wv_ref[...], preferred_element_type=jnp.float32) + bv_ref[...]


def _rowmax_kernel(q_ref, k_ref, m_ref):
    Q = q_ref[0]
    K = k_ref[0]
    T = Q.shape[0]
    nq = Q * jax.lax.rsqrt(jnp.maximum(jnp.sum(Q * Q, axis=-1, keepdims=True), 1e-12))
    nk = K * jax.lax.rsqrt(jnp.maximum(jnp.sum(K * K, axis=-1, keepdims=True), 1e-12))
    m = jnp.full((1, T), -jnp.inf, dtype=jnp.float32)
    for c in range(T // KEY_CHUNK):
        kc = nk[c * KEY_CHUNK:(c + 1) * KEY_CHUNK]
        st = jax.lax.dot_general(kc, nq, (((1,), (1,)), ((), ())),
                                 preferred_element_type=jnp.float32)
        m = jnp.maximum(m, jnp.max(st, axis=0, keepdims=True))
    m_ref[0] = m


def _topk_kernel(m_ref, idx_ref):
    v0 = m_ref[...]
    BH, T = v0.shape
    iota = jax.lax.broadcasted_iota(jnp.int32, (BH, T), 1)
    rank_iota = jax.lax.broadcasted_iota(jnp.int32, (BH, U), 1)
    acc0 = jnp.zeros((BH, U), dtype=jnp.int32)

    def body(r, carry):
        v, acc = carry
        mx = jnp.max(v, axis=1, keepdims=True)
        oh = v == mx
        idx = jnp.max(jnp.where(oh, iota, -1), axis=1, keepdims=True)
        acc = jnp.where(rank_iota == r, idx, acc)
        return jnp.where(iota == idx, -jnp.inf, v), acc

    _, acc = jax.lax.fori_loop(0, U, body, (v0, acc0))
    idx_ref[...] = acc


def _attn_kernel(idx_ref, q_ref, k_ref, v_ref, o_ref):
    Q = q_ref[0]
    K = k_ref[0]
    V = v_ref[0]
    T = Q.shape[0]
    idx_row = idx_ref[0]
    idx_col = jnp.reshape(idx_row, (U, 1))

    iota_row = jax.lax.broadcasted_iota(jnp.int32, (1, T), 1)
    iota_col = jax.lax.broadcasted_iota(jnp.int32, (T, 1), 0)
    P = (idx_col == iota_row).astype(jnp.float32)
    Pt = (iota_col == idx_row).astype(jnp.float32)

    q_sel = jnp.dot(P, Q, preferred_element_type=jnp.float32)
    s = jax.lax.dot_general(q_sel, K, (((1,), (1,)), ((), ())),
                            preferred_element_type=jnp.float32) * 0.125
    s = s - jnp.max(s, axis=-1, keepdims=True)
    e = jnp.exp(s)
    a = e / jnp.sum(e, axis=-1, keepdims=True)
    out = jnp.dot(a, V, preferred_element_type=jnp.float32)
    o_ref[0] = jnp.dot(Pt, out, preferred_element_type=jnp.float32)


def _proj_kernel(o_ref, wo_ref, bo_ref, y_ref):
    y_ref[...] = jnp.dot(o_ref[...], wo_ref[...], preferred_element_type=jnp.float32) + bo_ref[...]


def kernel(x, Wq, bq, Wk, bk, Wv, bv, Wo, bo):
    B, T, d_model = x.shape
    H = NUM_HEADS
    d_k = d_model // H
    BT = B * T
    BH = B * H

    x2 = x.reshape(BT, d_model)
    bq2 = bq.reshape(1, d_model)
    bk2 = bk.reshape(1, d_model)
    bv2 = bv.reshape(1, d_model)
    bo2 = bo.reshape(1, d_model)

    n_row = BT // ROW_BLOCK
    row_spec = pl.BlockSpec((ROW_BLOCK, d_model), lambda i: (i, 0))
    w_spec = pl.BlockSpec((d_model, d_model), lambda i: (0, 0))
    b_spec = pl.BlockSpec((1, d_model), lambda i: (0, 0))

    q, k, v = pl.pallas_call(
        _qkv_kernel,
        grid=(n_row,),
        in_specs=[row_spec, w_spec, w_spec, w_spec, b_spec, b_spec, b_spec],
        out_specs=[row_spec, row_spec, row_spec],
        out_shape=[jax.ShapeDtypeStruct((BT, d_model), jnp.float32)] * 3,
    )(x2, Wq, Wk, Wv, bq2, bk2, bv2)

    q3 = q.reshape(BH, T, d_k)
    k3 = k.reshape(BH, T, d_k)
    v3 = v.reshape(BH, T, d_k)

    bh_spec = pl.BlockSpec((1, T, d_k), lambda j: (j, 0, 0))
    m_spec = pl.BlockSpec((1, 1, T), lambda j: (j, 0, 0))
    m = pl.pallas_call(
        _rowmax_kernel,
        grid=(BH,),
        in_specs=[bh_spec, bh_spec],
        out_specs=m_spec,
        out_shape=jax.ShapeDtypeStruct((BH, 1, T), jnp.float32),
    )(q3, k3)
    m = m.reshape(BH, T)

    top_idx = pl.pallas_call(
        _topk_kernel,
        in_specs=[pl.BlockSpec((BH, T), lambda: (0, 0))],
        out_specs=pl.BlockSpec((BH, U), lambda: (0, 0)),
        out_shape=jax.ShapeDtypeStruct((BH, U), jnp.int32),
    )(m)

    idx3 = top_idx.reshape(BH, 1, U)
    idx_spec = pl.BlockSpec((1, 1, U), lambda j: (j, 0, 0))
    o = pl.pallas_call(
        _attn_kernel,
        grid=(BH,),
        in_specs=[idx_spec, bh_spec, bh_spec, bh_spec],
        out_specs=bh_spec,
        out_shape=jax.ShapeDtypeStruct((BH, T, d_k), jnp.float32),
    )(idx3, q3, k3, v3)

    o2 = o.reshape(BT, d_model)
    y = pl.pallas_call(
        _proj_kernel,
        grid=(n_row,),
        in_specs=[row_spec, w_spec, b_spec],
        out_specs=row_spec,
        out_shape=jax.ShapeDtypeStruct((BT, d_model), jnp.float32),
    )(o2, Wo, bo2)

    return y.reshape(B, T, d_model)
